# baseline (speedup 1.0000x reference)
import functools

import jax
import jax.numpy as jnp
from jax import lax
from jax.experimental import pallas as pl
from jax.experimental.pallas import tpu as pltpu
from jax.experimental.pallas import tpu_sc as plsc

N = 10000
NPAD = 10240
D = 128
DO = 16
NC, NS = 2, 16
NW = NC * NS
CH = 128
EK = 2048
CPT = 4
BR = 1024

_MESH = plsc.VectorSubcoreMesh(core_axis_name="c", subcore_axis_name="s")
_SC_PARAMS = pltpu.CompilerParams(
    use_tc_tiling_on_sc=False, needs_layout_passes=False)


def _zero_vmem_2d(ref, rows, cols):
    zc = cols // 16

    def body(i, _):
        r = i // zc
        k = i % zc
        ref[r, pl.ds(k * 16, 16)] = jnp.zeros((16,), jnp.float32)
        return 0

    lax.fori_loop(0, rows * zc, body, 0)


def _make_deg_kernel(ept):
    n_chunks = ept // CH
    grp = 16

    @functools.partial(
        pl.kernel,
        out_type=jax.ShapeDtypeStruct((NC * NPAD,), jnp.float32),
        mesh=_MESH,
        compiler_params=_SC_PARAMS,
        scratch_types=[
            pltpu.VMEM((n_chunks, CH), jnp.int32),
            pltpu.VMEM((CH,), jnp.float32),
            pltpu.VMEM((NPAD // NS,), jnp.float32),
            pltpu.VMEM_SHARED((NPAD,), jnp.float32),
            pltpu.SemaphoreType.DMA,
            pltpu.SemaphoreType.DMA,
        ],
    )
    def deg_kernel(dst_hbm, deg_hbm, idx_v, ones_v, stage_v, acc_sh,
                   sem_i, sem_s):
        c = lax.axis_index("c")
        s = lax.axis_index("s")
        wid = c * NS + s
        seg = NPAD // NS

        idx_dma = pltpu.async_copy(
            dst_hbm.at[pl.ds(wid * n_chunks, n_chunks)], idx_v, sem_i)

        def zbody(i, _):
            stage_v[pl.ds(i * 16, 16)] = jnp.zeros((16,), jnp.float32)
            return 0

        lax.fori_loop(0, seg // 16, zbody, 0)

        def obody(i, _):
            ones_v[pl.ds(i * 16, 16)] = jnp.ones((16,), jnp.float32)
            return 0

        lax.fori_loop(0, CH // 16, obody, 0)

        pltpu.sync_copy(stage_v, acc_sh.at[pl.ds(s * seg, seg)])
        plsc.subcore_barrier()
        idx_dma.wait()

        def group(g, _):
            def fire(j, _):
                pltpu.async_copy(
                    ones_v, acc_sh.at[idx_v.at[g * grp + j]], sem_s, add=True)
                return 0

            lax.fori_loop(0, grp, fire, 0)

            def drain(j, _):
                pltpu.make_async_copy(
                    ones_v, acc_sh.at[idx_v.at[0]], sem_s).wait()
                return 0

            lax.fori_loop(0, grp, drain, 0)
            return 0

        lax.fori_loop(0, n_chunks // grp, group, 0)
        plsc.subcore_barrier()

        pltpu.sync_copy(acc_sh.at[pl.ds(s * seg, seg)], stage_v)
        pltpu.sync_copy(stage_v, deg_hbm.at[pl.ds(c * NPAD + s * seg, seg)])

    return deg_kernel


def _make_colagg_kernel(epad):
    eps = epad // NC
    n_ck = eps // EK
    passes = D // (NS * CPT)

    @functools.partial(
        pl.kernel,
        out_type=jax.ShapeDtypeStruct((NC * D, NPAD), jnp.float32),
        mesh=_MESH,
        compiler_params=_SC_PARAMS,
        scratch_types=[
            pltpu.VMEM((CPT, NPAD), jnp.float32),
            pltpu.VMEM((CPT, NPAD), jnp.float32),
            pltpu.VMEM((EK,), jnp.int32),
            pltpu.VMEM((EK,), jnp.int32),
            pltpu.SemaphoreType.DMA,
            pltpu.SemaphoreType.DMA,
            pltpu.SemaphoreType.DMA,
        ],
    )
    def colagg_kernel(pk_hbm, hsT_hbm, accT_hbm,
                      tbl, acc, e0, e1, sem_t, sem0, sem1):
        c = lax.axis_index("c")
        s = lax.axis_index("s")
        ebase = c * eps

        def compute(buf):
            @plsc.parallel_loop(0, EK // 16, 1, unroll=8)
            def grp(g):
                ev = buf[pl.ds(g * 16, 16)]
                sv = ev & 0x3FFF
                dv = lax.shift_right_logical(ev, 14)
                for j in range(CPT):
                    jj = jnp.full((16,), j, jnp.int32)
                    v = plsc.load_gather(tbl, [jj, sv])
                    plsc.addupdate_scatter(acc, [jj, dv], v)

        for p in range(passes):
            row0 = p * (NS * CPT) + s * CPT
            tdma = pltpu.async_copy(
                hsT_hbm.at[pl.ds(row0, CPT)], tbl, sem_t)
            _zero_vmem_2d(acc, CPT, NPAD)
            tdma.wait()

            pltpu.async_copy(pk_hbm.at[pl.ds(ebase, EK)], e0, sem0)

            def pair(i, _):
                c0 = 2 * i
                pltpu.async_copy(
                    pk_hbm.at[pl.ds(ebase + (c0 + 1) * EK, EK)], e1, sem1)
                pltpu.make_async_copy(
                    pk_hbm.at[pl.ds(0, EK)], e0, sem0).wait()
                compute(e0)

                @pl.when(i < n_ck // 2 - 1)
                def _():
                    pltpu.async_copy(
                        pk_hbm.at[pl.ds(ebase + (c0 + 2) * EK, EK)], e0, sem0)

                pltpu.make_async_copy(
                    pk_hbm.at[pl.ds(0, EK)], e1, sem1).wait()
                compute(e1)
                return 0

            lax.fori_loop(0, n_ck // 2, pair, 0)

            pltpu.sync_copy(acc, accT_hbm.at[pl.ds(c * D + row0, CPT)])

    return colagg_kernel


def _make_scatter_kernel(ept, fd):
    n_chunks = ept // CH
    PH = 2
    pc = n_chunks // PH
    npairs = pc // 2
    seg = NPAD // NS

    @functools.partial(
        pl.kernel,
        out_type=jax.ShapeDtypeStruct((NC * NPAD, fd), jnp.float32),
        mesh=_MESH,
        compiler_params=_SC_PARAMS,
        scratch_types=[
            pltpu.VMEM((pc, CH), jnp.int32),
            pltpu.VMEM((pc, CH), jnp.int32),
            pltpu.VMEM((CH, fd), jnp.float32),
            pltpu.VMEM((CH, fd), jnp.float32),
            pltpu.VMEM_SHARED((NPAD, fd), jnp.float32),
            pltpu.SemaphoreType.DMA,
            pltpu.SemaphoreType.DMA,
            pltpu.SemaphoreType.DMA,
        ],
    )
    def scat_kernel(src_hbm, dst_hbm, feat_hbm, out_hbm,
                    si_v, di_v, rows0, rows1, acc_sh, sem_i, sem0, sem1):
        c = lax.axis_index("c")
        s = lax.axis_index("s")
        wid = c * NS + s

        si_dma = pltpu.async_copy(
            src_hbm.at[pl.ds(wid * n_chunks, pc)], si_v, sem_i)
        di_dma = pltpu.async_copy(
            dst_hbm.at[pl.ds(wid * n_chunks, pc)], di_v, sem_i)

        _zero_vmem_2d(rows0, CH, fd)
        nz = seg // CH

        def zc(i, _):
            pltpu.sync_copy(rows0, acc_sh.at[pl.ds(s * seg + i * CH, CH)])
            return 0

        lax.fori_loop(0, nz, zc, 0)
        plsc.subcore_barrier()

        for ph in range(PH):
            if ph > 0:
                pltpu.async_copy(
                    src_hbm.at[pl.ds(wid * n_chunks + ph * pc, pc)],
                    si_v, sem_i).wait()
                pltpu.async_copy(
                    dst_hbm.at[pl.ds(wid * n_chunks + ph * pc, pc)],
                    di_v, sem_i).wait()
            else:
                si_dma.wait()
                di_dma.wait()

            pltpu.async_copy(feat_hbm.at[si_v.at[0]], rows0, sem0)

            def pair(i, _):
                c0 = 2 * i
                pltpu.async_copy(feat_hbm.at[si_v.at[c0 + 1]], rows1, sem1)
                pltpu.make_async_copy(
                    feat_hbm.at[si_v.at[0]], rows0, sem0).wait()
                pltpu.sync_copy(rows0, acc_sh.at[di_v.at[c0]], add=True)

                @pl.when(i < npairs - 1)
                def _():
                    pltpu.async_copy(
                        feat_hbm.at[si_v.at[c0 + 2]], rows0, sem0)

                pltpu.make_async_copy(
                    feat_hbm.at[si_v.at[0]], rows1, sem1).wait()
                pltpu.sync_copy(rows1, acc_sh.at[di_v.at[c0 + 1]], add=True)
                return 0

            lax.fori_loop(0, npairs, pair, 0)

        plsc.subcore_barrier()

        def oc(i, _):
            pltpu.sync_copy(acc_sh.at[pl.ds(s * seg + i * CH, CH)], rows0)
            pltpu.sync_copy(
                rows0, out_hbm.at[pl.ds(c * NPAD + s * seg + i * CH, CH)])
            return 0

        lax.fori_loop(0, nz, oc, 0)

    return scat_kernel


def _tc_b(deg0t_ref, deg1t_ref, x_ref, w1_ref, hsT_ref, dinvT_ref):
    degT = deg0t_ref[...] + deg1t_ref[...] + 1.0
    dinvT = lax.rsqrt(degT)
    hT = lax.dot_general(
        w1_ref[...], x_ref[...],
        dimension_numbers=(((0,), (1,)), ((), ())),
        preferred_element_type=jnp.float32)
    hsT_ref[...] = hT * dinvT
    dinvT_ref[...] = dinvT


def _tc_d(a0_ref, a1_ref, hsT_ref, dinvT_ref, b1_ref, w2_ref, gs_ref):
    i = pl.program_id(0)
    dinvT = dinvT_ref[...]
    pre = a0_ref[...] + a1_ref[...] + hsT_ref[...]
    o = pre * dinvT + b1_ref[...]
    h1 = jnp.maximum(o, 0.0)
    g = lax.dot_general(
        h1, w2_ref[...],
        dimension_numbers=(((0,), (0,)), ((), ())),
        preferred_element_type=jnp.float32)
    dinv_col = jnp.transpose(dinvT)
    row = jax.lax.broadcasted_iota(jnp.int32, (BR, 1), 0) + i * BR
    gs_ref[...] = jnp.where(row < N, g * dinv_col, 0.0)


def _tc_f(a0_ref, a1_ref, gs_ref, dinvT_ref, b2_ref, out_ref):
    dinv = jnp.transpose(dinvT_ref[...])
    o = (a0_ref[...] + a1_ref[...] + gs_ref[...]) * dinv + b2_ref[...]
    o8 = o[:, :8]
    m = jnp.max(o8, axis=1, keepdims=True)
    e = jnp.exp(o8 - m)
    lse = jnp.log(jnp.sum(e, axis=1, keepdims=True))
    out_ref[...] = o8 - m - lse


def kernel(x, edge_index, W1, b1, W2, b2):
    E = edge_index.shape[1]
    ept = ((E + NW * 2 * CH - 1) // (NW * 2 * CH)) * (2 * CH)
    epad = ept * NW
    pad = epad - E
    n_chunks = ept // CH

    src = jnp.concatenate([edge_index[0], jnp.full((pad,), N, jnp.int32)])
    dst = jnp.concatenate([edge_index[1], jnp.full((pad,), N, jnp.int32)])
    packed = src | (dst << 14)
    src2 = src.reshape(NW * n_chunks, CH)
    dst2 = dst.reshape(NW * n_chunks, CH)

    x_pad = jnp.pad(x, ((0, NPAD - N), (0, 0)))
    w2p = jnp.pad(W2, ((0, 0), (0, DO - W2.shape[1])))
    b1c = b1.reshape(D, 1)
    b2r = jnp.pad(b2, (0, DO - b2.shape[0])).reshape(1, DO)

    deg = _make_deg_kernel(ept)(dst2)
    deg0t = deg[:NPAD].reshape(1, NPAD)
    deg1t = deg[NPAD:].reshape(1, NPAD)

    grid = NPAD // BR
    hsT, dinvT = pl.pallas_call(
        _tc_b,
        grid=(grid,),
        in_specs=[
            pl.BlockSpec((1, BR), lambda i: (0, i)),
            pl.BlockSpec((1, BR), lambda i: (0, i)),
            pl.BlockSpec((BR, D), lambda i: (i, 0)),
            pl.BlockSpec((D, D), lambda i: (0, 0)),
        ],
        out_specs=[
            pl.BlockSpec((D, BR), lambda i: (0, i)),
            pl.BlockSpec((1, BR), lambda i: (0, i)),
        ],
        out_shape=[
            jax.ShapeDtypeStruct((D, NPAD), jnp.float32),
            jax.ShapeDtypeStruct((1, NPAD), jnp.float32),
        ],
    )(deg0t, deg1t, x_pad, W1)

    accT = _make_colagg_kernel(epad)(packed, hsT)
    a0T = accT[:D]
    a1T = accT[D:]

    gs = pl.pallas_call(
        _tc_d,
        grid=(grid,),
        in_specs=[
            pl.BlockSpec((D, BR), lambda i: (0, i)),
            pl.BlockSpec((D, BR), lambda i: (0, i)),
            pl.BlockSpec((D, BR), lambda i: (0, i)),
            pl.BlockSpec((1, BR), lambda i: (0, i)),
            pl.BlockSpec((D, 1), lambda i: (0, 0)),
            pl.BlockSpec((D, DO), lambda i: (0, 0)),
        ],
        out_specs=pl.BlockSpec((BR, DO), lambda i: (i, 0)),
        out_shape=jax.ShapeDtypeStruct((NPAD, DO), jnp.float32),
    )(a0T, a1T, hsT, dinvT, b1c, w2p)

    acc2 = _make_scatter_kernel(ept, DO)(src2, dst2, gs)
    a20 = acc2[:NPAD]
    a21 = acc2[NPAD:]

    out = pl.pallas_call(
        _tc_f,
        grid=(grid,),
        in_specs=[
            pl.BlockSpec((BR, DO), lambda i: (i, 0)),
            pl.BlockSpec((BR, DO), lambda i: (i, 0)),
            pl.BlockSpec((BR, DO), lambda i: (i, 0)),
            pl.BlockSpec((1, BR), lambda i: (0, i)),
            pl.BlockSpec((1, DO), lambda i: (0, 0)),
        ],
        out_specs=pl.BlockSpec((BR, 8), lambda i: (i, 0)),
        out_shape=jax.ShapeDtypeStruct((NPAD, 8), jnp.float32),
    )(a20, a21, gs, dinvT, b2r)

    return out[:N]


# speedup vs baseline: 1.0519x; 1.0519x over previous
import functools

import jax
import jax.numpy as jnp
from jax import lax
from jax.experimental import pallas as pl
from jax.experimental.pallas import tpu as pltpu
from jax.experimental.pallas import tpu_sc as plsc

N = 10000
NPAD = 10240
D = 128
DO = 16
NC, NS = 2, 16
NW = NC * NS
CH = 128
EK = 512
CPT = 8
BR = 1024

_MESH = plsc.VectorSubcoreMesh(core_axis_name="c", subcore_axis_name="s")
_SC_PARAMS = pltpu.CompilerParams(
    use_tc_tiling_on_sc=False, needs_layout_passes=False)


def _zero_vmem_2d(ref, rows, cols):
    zc = cols // 16

    def body(i, _):
        r = i // zc
        k = i % zc
        ref[r, pl.ds(k * 16, 16)] = jnp.zeros((16,), jnp.float32)
        return 0

    lax.fori_loop(0, rows * zc, body, 0)


def _make_deg_kernel(ept):
    n_chunks = ept // CH
    grp = 16

    @functools.partial(
        pl.kernel,
        out_type=jax.ShapeDtypeStruct((NC * NPAD,), jnp.float32),
        mesh=_MESH,
        compiler_params=_SC_PARAMS,
        scratch_types=[
            pltpu.VMEM((n_chunks, CH), jnp.int32),
            pltpu.VMEM((CH,), jnp.float32),
            pltpu.VMEM((NPAD // NS,), jnp.float32),
            pltpu.VMEM_SHARED((NPAD,), jnp.float32),
            pltpu.SemaphoreType.DMA,
            pltpu.SemaphoreType.DMA,
        ],
    )
    def deg_kernel(dst_hbm, deg_hbm, idx_v, ones_v, stage_v, acc_sh,
                   sem_i, sem_s):
        c = lax.axis_index("c")
        s = lax.axis_index("s")
        wid = c * NS + s
        seg = NPAD // NS

        idx_dma = pltpu.async_copy(
            dst_hbm.at[pl.ds(wid * n_chunks, n_chunks)], idx_v, sem_i)

        def zbody(i, _):
            stage_v[pl.ds(i * 16, 16)] = jnp.zeros((16,), jnp.float32)
            return 0

        lax.fori_loop(0, seg // 16, zbody, 0)

        def obody(i, _):
            ones_v[pl.ds(i * 16, 16)] = jnp.ones((16,), jnp.float32)
            return 0

        lax.fori_loop(0, CH // 16, obody, 0)

        pltpu.sync_copy(stage_v, acc_sh.at[pl.ds(s * seg, seg)])
        plsc.subcore_barrier()
        idx_dma.wait()

        def group(g, _):
            def fire(j, _):
                pltpu.async_copy(
                    ones_v, acc_sh.at[idx_v.at[g * grp + j]], sem_s, add=True)
                return 0

            lax.fori_loop(0, grp, fire, 0)

            def drain(j, _):
                pltpu.make_async_copy(
                    ones_v, acc_sh.at[idx_v.at[0]], sem_s).wait()
                return 0

            lax.fori_loop(0, grp, drain, 0)
            return 0

        lax.fori_loop(0, n_chunks // grp, group, 0)
        plsc.subcore_barrier()

        pltpu.sync_copy(acc_sh.at[pl.ds(s * seg, seg)], stage_v)
        pltpu.sync_copy(stage_v, deg_hbm.at[pl.ds(c * NPAD + s * seg, seg)])

    return deg_kernel


def _make_colagg_kernel(epad):
    eps = epad // NC
    n_ck = eps // EK
    CP2 = CPT // 2

    @functools.partial(
        pl.kernel,
        out_type=jax.ShapeDtypeStruct((NC * D, NPAD), jnp.float32),
        mesh=_MESH,
        compiler_params=_SC_PARAMS,
        scratch_types=[
            pltpu.VMEM((CP2, NPAD), jnp.int32),
            pltpu.VMEM((CPT, NPAD), jnp.float32),
            pltpu.VMEM((EK,), jnp.int32),
            pltpu.VMEM((EK,), jnp.int32),
            pltpu.SemaphoreType.DMA,
            pltpu.SemaphoreType.DMA,
            pltpu.SemaphoreType.DMA,
        ],
    )
    def colagg_kernel(pk_hbm, hsP_hbm, accT_hbm,
                      tbl, acc, e0, e1, sem_t, sem0, sem1):
        c = lax.axis_index("c")
        s = lax.axis_index("s")
        ebase = c * eps

        def compute(buf):
            @plsc.parallel_loop(0, EK // 16, 1, unroll=4)
            def grp(g):
                ev = buf[pl.ds(g * 16, 16)]
                sv = ev & 0x3FFF
                dv = lax.shift_right_logical(ev, 14)
                for j in range(CP2):
                    jj = jnp.full((16,), j, jnp.int32)
                    vp = plsc.load_gather(tbl, [jj, sv])
                    lo = plsc.bitcast(lax.shift_left(vp, 16), jnp.float32)
                    hi = plsc.bitcast(vp & jnp.int32(-65536), jnp.float32)
                    j1 = jnp.full((16,), CP2 + j, jnp.int32)
                    plsc.addupdate_scatter(acc, [jj, dv], lo)
                    plsc.addupdate_scatter(acc, [j1, dv], hi)

        tdma = pltpu.async_copy(
            hsP_hbm.at[pl.ds(s * CP2, CP2)], tbl, sem_t)
        _zero_vmem_2d(acc, CPT, NPAD)
        tdma.wait()

        pltpu.async_copy(pk_hbm.at[pl.ds(ebase, EK)], e0, sem0)

        def pair(i, _):
            c0 = 2 * i
            pltpu.async_copy(
                pk_hbm.at[pl.ds(ebase + (c0 + 1) * EK, EK)], e1, sem1)
            pltpu.make_async_copy(
                pk_hbm.at[pl.ds(0, EK)], e0, sem0).wait()
            compute(e0)

            @pl.when(i < n_ck // 2 - 1)
            def _():
                pltpu.async_copy(
                    pk_hbm.at[pl.ds(ebase + (c0 + 2) * EK, EK)], e0, sem0)

            pltpu.make_async_copy(
                pk_hbm.at[pl.ds(0, EK)], e1, sem1).wait()
            compute(e1)
            return 0

        lax.fori_loop(0, n_ck // 2, pair, 0)

        pltpu.sync_copy(
            acc.at[pl.ds(0, CP2)], accT_hbm.at[pl.ds(c * D + s * CP2, CP2)])
        pltpu.sync_copy(
            acc.at[pl.ds(CP2, CP2)],
            accT_hbm.at[pl.ds(c * D + D // 2 + s * CP2, CP2)])

    return colagg_kernel


def _make_scatter_kernel(ept, fd):
    n_chunks = ept // CH
    PH = 2
    pc = n_chunks // PH
    npairs = pc // 2
    seg = NPAD // NS

    @functools.partial(
        pl.kernel,
        out_type=jax.ShapeDtypeStruct((NC * NPAD, fd), jnp.float32),
        mesh=_MESH,
        compiler_params=_SC_PARAMS,
        scratch_types=[
            pltpu.VMEM((pc, CH), jnp.int32),
            pltpu.VMEM((pc, CH), jnp.int32),
            pltpu.VMEM((CH, fd), jnp.float32),
            pltpu.VMEM((CH, fd), jnp.float32),
            pltpu.VMEM_SHARED((NPAD, fd), jnp.float32),
            pltpu.SemaphoreType.DMA,
            pltpu.SemaphoreType.DMA,
            pltpu.SemaphoreType.DMA,
        ],
    )
    def scat_kernel(src_hbm, dst_hbm, feat_hbm, out_hbm,
                    si_v, di_v, rows0, rows1, acc_sh, sem_i, sem0, sem1):
        c = lax.axis_index("c")
        s = lax.axis_index("s")
        wid = c * NS + s

        si_dma = pltpu.async_copy(
            src_hbm.at[pl.ds(wid * n_chunks, pc)], si_v, sem_i)
        di_dma = pltpu.async_copy(
            dst_hbm.at[pl.ds(wid * n_chunks, pc)], di_v, sem_i)

        _zero_vmem_2d(rows0, CH, fd)
        nz = seg // CH

        def zc(i, _):
            pltpu.sync_copy(rows0, acc_sh.at[pl.ds(s * seg + i * CH, CH)])
            return 0

        lax.fori_loop(0, nz, zc, 0)
        plsc.subcore_barrier()

        for ph in range(PH):
            if ph > 0:
                pltpu.async_copy(
                    src_hbm.at[pl.ds(wid * n_chunks + ph * pc, pc)],
                    si_v, sem_i).wait()
                pltpu.async_copy(
                    dst_hbm.at[pl.ds(wid * n_chunks + ph * pc, pc)],
                    di_v, sem_i).wait()
            else:
                si_dma.wait()
                di_dma.wait()

            pltpu.async_copy(feat_hbm.at[si_v.at[0]], rows0, sem0)

            def pair(i, _):
                c0 = 2 * i
                pltpu.async_copy(feat_hbm.at[si_v.at[c0 + 1]], rows1, sem1)
                pltpu.make_async_copy(
                    feat_hbm.at[si_v.at[0]], rows0, sem0).wait()
                pltpu.sync_copy(rows0, acc_sh.at[di_v.at[c0]], add=True)

                @pl.when(i < npairs - 1)
                def _():
                    pltpu.async_copy(
                        feat_hbm.at[si_v.at[c0 + 2]], rows0, sem0)

                pltpu.make_async_copy(
                    feat_hbm.at[si_v.at[0]], rows1, sem1).wait()
                pltpu.sync_copy(rows1, acc_sh.at[di_v.at[c0 + 1]], add=True)
                return 0

            lax.fori_loop(0, npairs, pair, 0)

        plsc.subcore_barrier()

        def oc(i, _):
            pltpu.sync_copy(acc_sh.at[pl.ds(s * seg + i * CH, CH)], rows0)
            pltpu.sync_copy(
                rows0, out_hbm.at[pl.ds(c * NPAD + s * seg + i * CH, CH)])
            return 0

        lax.fori_loop(0, nz, oc, 0)

    return scat_kernel


def _tc_b(deg0t_ref, deg1t_ref, x_ref, w1_ref, hsT_ref, hsP_ref, dinvT_ref):
    degT = deg0t_ref[...] + deg1t_ref[...] + 1.0
    dinvT = lax.rsqrt(degT)
    hT = lax.dot_general(
        w1_ref[...], x_ref[...],
        dimension_numbers=(((0,), (1,)), ((), ())),
        preferred_element_type=jnp.float32)
    hsT = hT * dinvT
    hsT_ref[...] = hsT
    dinvT_ref[...] = dinvT
    lo = lax.bitcast_convert_type(
        hsT[:D // 2].astype(jnp.bfloat16), jnp.uint16).astype(jnp.uint32)
    hi = lax.bitcast_convert_type(
        hsT[D // 2:].astype(jnp.bfloat16), jnp.uint16).astype(jnp.uint32)
    hsP_ref[...] = lax.bitcast_convert_type((hi << 16) | lo, jnp.int32)


def _tc_d(a0_ref, a1_ref, hsT_ref, dinvT_ref, b1_ref, w2_ref, gs_ref):
    i = pl.program_id(0)
    dinvT = dinvT_ref[...]
    pre = a0_ref[...] + a1_ref[...] + hsT_ref[...]
    o = pre * dinvT + b1_ref[...]
    h1 = jnp.maximum(o, 0.0)
    g = lax.dot_general(
        h1, w2_ref[...],
        dimension_numbers=(((0,), (0,)), ((), ())),
        preferred_element_type=jnp.float32)
    dinv_col = jnp.transpose(dinvT)
    row = jax.lax.broadcasted_iota(jnp.int32, (BR, 1), 0) + i * BR
    gs_ref[...] = jnp.where(row < N, g * dinv_col, 0.0)


def _tc_f(a0_ref, a1_ref, gs_ref, dinvT_ref, b2_ref, out_ref):
    dinv = jnp.transpose(dinvT_ref[...])
    o = (a0_ref[...] + a1_ref[...] + gs_ref[...]) * dinv + b2_ref[...]
    o8 = o[:, :8]
    m = jnp.max(o8, axis=1, keepdims=True)
    e = jnp.exp(o8 - m)
    lse = jnp.log(jnp.sum(e, axis=1, keepdims=True))
    out_ref[...] = o8 - m - lse


def kernel(x, edge_index, W1, b1, W2, b2):
    E = edge_index.shape[1]
    ept = ((E + NW * 2 * CH - 1) // (NW * 2 * CH)) * (2 * CH)
    epad = ept * NW
    pad = epad - E
    n_chunks = ept // CH

    src = jnp.concatenate([edge_index[0], jnp.full((pad,), N, jnp.int32)])
    dst = jnp.concatenate([edge_index[1], jnp.full((pad,), N, jnp.int32)])
    packed = src | (dst << 14)
    src2 = src.reshape(NW * n_chunks, CH)
    dst2 = dst.reshape(NW * n_chunks, CH)

    x_pad = jnp.pad(x, ((0, NPAD - N), (0, 0)))
    w2p = jnp.pad(W2, ((0, 0), (0, DO - W2.shape[1])))
    b1c = b1.reshape(D, 1)
    b2r = jnp.pad(b2, (0, DO - b2.shape[0])).reshape(1, DO)

    deg = _make_deg_kernel(ept)(dst2)
    deg0t = deg[:NPAD].reshape(1, NPAD)
    deg1t = deg[NPAD:].reshape(1, NPAD)

    grid = NPAD // BR
    hsT, hsP, dinvT = pl.pallas_call(
        _tc_b,
        grid=(grid,),
        in_specs=[
            pl.BlockSpec((1, BR), lambda i: (0, i)),
            pl.BlockSpec((1, BR), lambda i: (0, i)),
            pl.BlockSpec((BR, D), lambda i: (i, 0)),
            pl.BlockSpec((D, D), lambda i: (0, 0)),
        ],
        out_specs=[
            pl.BlockSpec((D, BR), lambda i: (0, i)),
            pl.BlockSpec((D // 2, BR), lambda i: (0, i)),
            pl.BlockSpec((1, BR), lambda i: (0, i)),
        ],
        out_shape=[
            jax.ShapeDtypeStruct((D, NPAD), jnp.float32),
            jax.ShapeDtypeStruct((D // 2, NPAD), jnp.int32),
            jax.ShapeDtypeStruct((1, NPAD), jnp.float32),
        ],
    )(deg0t, deg1t, x_pad, W1)

    accT = _make_colagg_kernel(epad)(packed, hsP)
    a0T = accT[:D]
    a1T = accT[D:]

    gs = pl.pallas_call(
        _tc_d,
        grid=(grid,),
        in_specs=[
            pl.BlockSpec((D, BR), lambda i: (0, i)),
            pl.BlockSpec((D, BR), lambda i: (0, i)),
            pl.BlockSpec((D, BR), lambda i: (0, i)),
            pl.BlockSpec((1, BR), lambda i: (0, i)),
            pl.BlockSpec((D, 1), lambda i: (0, 0)),
            pl.BlockSpec((D, DO), lambda i: (0, 0)),
        ],
        out_specs=pl.BlockSpec((BR, DO), lambda i: (i, 0)),
        out_shape=jax.ShapeDtypeStruct((NPAD, DO), jnp.float32),
    )(a0T, a1T, hsT, dinvT, b1c, w2p)

    acc2 = _make_scatter_kernel(ept, DO)(src2, dst2, gs)
    a20 = acc2[:NPAD]
    a21 = acc2[NPAD:]

    out = pl.pallas_call(
        _tc_f,
        grid=(grid,),
        in_specs=[
            pl.BlockSpec((BR, DO), lambda i: (i, 0)),
            pl.BlockSpec((BR, DO), lambda i: (i, 0)),
            pl.BlockSpec((BR, DO), lambda i: (i, 0)),
            pl.BlockSpec((1, BR), lambda i: (0, i)),
            pl.BlockSpec((1, DO), lambda i: (0, 0)),
        ],
        out_specs=pl.BlockSpec((BR, 8), lambda i: (i, 0)),
        out_shape=jax.ShapeDtypeStruct((NPAD, 8), jnp.float32),
    )(a20, a21, gs, dinvT, b2r)

    return out[:N]


# speedup vs baseline: 1.1234x; 1.0679x over previous
import functools

import jax
import jax.numpy as jnp
from jax import lax
from jax.experimental import pallas as pl
from jax.experimental.pallas import tpu as pltpu
from jax.experimental.pallas import tpu_sc as plsc

N = 10000
NPAD = 10240
D = 128
NC, NS = 2, 16
NW = NC * NS
CH = 128
EK = 512
CPT = 8
BR = 1024

_MESH = plsc.VectorSubcoreMesh(core_axis_name="c", subcore_axis_name="s")
_SC_PARAMS = pltpu.CompilerParams(
    use_tc_tiling_on_sc=False, needs_layout_passes=False)


def _zero_vmem_2d(ref, rows, cols):
    zc = cols // 16

    def body(i, _):
        r = i // zc
        k = i % zc
        ref[r, pl.ds(k * 16, 16)] = jnp.zeros((16,), jnp.float32)
        return 0

    lax.fori_loop(0, rows * zc, body, 0)


def _make_deg_kernel(ept):
    n_chunks = ept // CH
    grp = 16

    @functools.partial(
        pl.kernel,
        out_type=jax.ShapeDtypeStruct((NC * NPAD,), jnp.float32),
        mesh=_MESH,
        compiler_params=_SC_PARAMS,
        scratch_types=[
            pltpu.VMEM((n_chunks, CH), jnp.int32),
            pltpu.VMEM((CH,), jnp.float32),
            pltpu.VMEM((NPAD // NS,), jnp.float32),
            pltpu.VMEM_SHARED((NPAD,), jnp.float32),
            pltpu.SemaphoreType.DMA,
            pltpu.SemaphoreType.DMA,
        ],
    )
    def deg_kernel(dst_hbm, deg_hbm, idx_v, ones_v, stage_v, acc_sh,
                   sem_i, sem_s):
        c = lax.axis_index("c")
        s = lax.axis_index("s")
        wid = c * NS + s
        seg = NPAD // NS

        idx_dma = pltpu.async_copy(
            dst_hbm.at[pl.ds(wid * n_chunks, n_chunks)], idx_v, sem_i)

        def zbody(i, _):
            stage_v[pl.ds(i * 16, 16)] = jnp.zeros((16,), jnp.float32)
            return 0

        lax.fori_loop(0, seg // 16, zbody, 0)

        def obody(i, _):
            ones_v[pl.ds(i * 16, 16)] = jnp.ones((16,), jnp.float32)
            return 0

        lax.fori_loop(0, CH // 16, obody, 0)

        pltpu.sync_copy(stage_v, acc_sh.at[pl.ds(s * seg, seg)])
        plsc.subcore_barrier()
        idx_dma.wait()

        def group(g, _):
            def fire(j, _):
                pltpu.async_copy(
                    ones_v, acc_sh.at[idx_v.at[g * grp + j]], sem_s, add=True)
                return 0

            lax.fori_loop(0, grp, fire, 0)

            def drain(j, _):
                pltpu.make_async_copy(
                    ones_v, acc_sh.at[idx_v.at[0]], sem_s).wait()
                return 0

            lax.fori_loop(0, grp, drain, 0)
            return 0

        lax.fori_loop(0, n_chunks // grp, group, 0)
        plsc.subcore_barrier()

        pltpu.sync_copy(acc_sh.at[pl.ds(s * seg, seg)], stage_v)
        pltpu.sync_copy(stage_v, deg_hbm.at[pl.ds(c * NPAD + s * seg, seg)])

    return deg_kernel


def _make_colagg_kernel(epad):
    eps = epad // NC
    n_ck = eps // EK
    CP2 = CPT // 2

    @functools.partial(
        pl.kernel,
        out_type=jax.ShapeDtypeStruct((NC * D, NPAD), jnp.float32),
        mesh=_MESH,
        compiler_params=_SC_PARAMS,
        scratch_types=[
            pltpu.VMEM((CP2, NPAD), jnp.int32),
            pltpu.VMEM((CPT, NPAD), jnp.float32),
            pltpu.VMEM((EK,), jnp.int32),
            pltpu.VMEM((EK,), jnp.int32),
            pltpu.SemaphoreType.DMA,
            pltpu.SemaphoreType.DMA,
            pltpu.SemaphoreType.DMA,
        ],
    )
    def colagg_kernel(pk_hbm, hsP_hbm, accT_hbm,
                      tbl, acc, e0, e1, sem_t, sem0, sem1):
        c = lax.axis_index("c")
        s = lax.axis_index("s")
        ebase = c * eps

        def compute(buf):
            @plsc.parallel_loop(0, EK // 16, 1, unroll=4)
            def grp(g):
                ev = buf[pl.ds(g * 16, 16)]
                sv = ev & 0x3FFF
                dv = lax.shift_right_logical(ev, 14)
                for j in range(CP2):
                    jj = jnp.full((16,), j, jnp.int32)
                    vp = plsc.load_gather(tbl, [jj, sv])
                    lo = plsc.bitcast(lax.shift_left(vp, 16), jnp.float32)
                    hi = plsc.bitcast(vp & jnp.int32(-65536), jnp.float32)
                    j1 = jnp.full((16,), CP2 + j, jnp.int32)
                    plsc.addupdate_scatter(acc, [jj, dv], lo)
                    plsc.addupdate_scatter(acc, [j1, dv], hi)

        tdma = pltpu.async_copy(
            hsP_hbm.at[pl.ds(s * CP2, CP2)], tbl, sem_t)
        _zero_vmem_2d(acc, CPT, NPAD)
        tdma.wait()

        pltpu.async_copy(pk_hbm.at[pl.ds(ebase, EK)], e0, sem0)

        def pair(i, _):
            c0 = 2 * i
            pltpu.async_copy(
                pk_hbm.at[pl.ds(ebase + (c0 + 1) * EK, EK)], e1, sem1)
            pltpu.make_async_copy(
                pk_hbm.at[pl.ds(0, EK)], e0, sem0).wait()
            compute(e0)

            @pl.when(i < n_ck // 2 - 1)
            def _():
                pltpu.async_copy(
                    pk_hbm.at[pl.ds(ebase + (c0 + 2) * EK, EK)], e0, sem0)

            pltpu.make_async_copy(
                pk_hbm.at[pl.ds(0, EK)], e1, sem1).wait()
            compute(e1)
            return 0

        lax.fori_loop(0, n_ck // 2, pair, 0)

        pltpu.sync_copy(
            acc.at[pl.ds(0, CP2)], accT_hbm.at[pl.ds(c * D + s * CP2, CP2)])
        pltpu.sync_copy(
            acc.at[pl.ds(CP2, CP2)],
            accT_hbm.at[pl.ds(c * D + D // 2 + s * CP2, CP2)])

    return colagg_kernel


def _make_colagg2_kernel(epad):
    eq = epad // 4
    n_ck = eq // EK

    @functools.partial(
        pl.kernel,
        out_type=jax.ShapeDtypeStruct((4 * 8, NPAD), jnp.float32),
        mesh=_MESH,
        compiler_params=_SC_PARAMS,
        scratch_types=[
            pltpu.VMEM((NPAD,), jnp.float32),
            pltpu.VMEM((NPAD,), jnp.float32),
            pltpu.VMEM((EK,), jnp.int32),
            pltpu.VMEM((EK,), jnp.int32),
            pltpu.SemaphoreType.DMA,
            pltpu.SemaphoreType.DMA,
            pltpu.SemaphoreType.DMA,
        ],
    )
    def colagg2_kernel(pk_hbm, gsT_hbm, out_hbm,
                       tbl, acc, e0, e1, sem_t, sem0, sem1):
        c = lax.axis_index("c")
        s = lax.axis_index("s")
        wid = c * NS + s
        col = wid % 8
        q = wid // 8
        ebase = q * eq

        def compute(buf):
            @plsc.parallel_loop(0, EK // 16, 1, unroll=4)
            def grp(g):
                ev = buf[pl.ds(g * 16, 16)]
                sv = ev & 0x3FFF
                dv = lax.shift_right_logical(ev, 14)
                v = plsc.load_gather(tbl, [sv])
                plsc.addupdate_scatter(acc, [dv], v)

        tdma = pltpu.async_copy(gsT_hbm.at[col], tbl, sem_t)

        def zbody(i, _):
            acc[pl.ds(i * 16, 16)] = jnp.zeros((16,), jnp.float32)
            return 0

        lax.fori_loop(0, NPAD // 16, zbody, 0)
        tdma.wait()

        pltpu.async_copy(pk_hbm.at[pl.ds(ebase, EK)], e0, sem0)

        def pair(i, _):
            c0 = 2 * i
            pltpu.async_copy(
                pk_hbm.at[pl.ds(ebase + (c0 + 1) * EK, EK)], e1, sem1)
            pltpu.make_async_copy(
                pk_hbm.at[pl.ds(0, EK)], e0, sem0).wait()
            compute(e0)

            @pl.when(i < n_ck // 2 - 1)
            def _():
                pltpu.async_copy(
                    pk_hbm.at[pl.ds(ebase + (c0 + 2) * EK, EK)], e0, sem0)

            pltpu.make_async_copy(
                pk_hbm.at[pl.ds(0, EK)], e1, sem1).wait()
            compute(e1)
            return 0

        lax.fori_loop(0, n_ck // 2, pair, 0)

        pltpu.sync_copy(acc, out_hbm.at[q * 8 + col])

    return colagg2_kernel


def _tc_b(deg0t_ref, deg1t_ref, x_ref, w1_ref, hsT_ref, hsP_ref, dinvT_ref):
    degT = deg0t_ref[...] + deg1t_ref[...] + 1.0
    dinvT = lax.rsqrt(degT)
    hT = lax.dot_general(
        w1_ref[...], x_ref[...],
        dimension_numbers=(((0,), (1,)), ((), ())),
        preferred_element_type=jnp.float32)
    hsT = hT * dinvT
    hsT_ref[...] = hsT
    dinvT_ref[...] = dinvT
    lo = lax.bitcast_convert_type(
        hsT[:D // 2].astype(jnp.bfloat16), jnp.uint16).astype(jnp.uint32)
    hi = lax.bitcast_convert_type(
        hsT[D // 2:].astype(jnp.bfloat16), jnp.uint16).astype(jnp.uint32)
    hsP_ref[...] = lax.bitcast_convert_type((hi << 16) | lo, jnp.int32)


def _tc_d(a0_ref, a1_ref, hsT_ref, dinvT_ref, b1_ref, w2_ref, gsT_ref):
    i = pl.program_id(0)
    dinvT = dinvT_ref[...]
    pre = a0_ref[...] + a1_ref[...] + hsT_ref[...]
    o = pre * dinvT + b1_ref[...]
    h1 = jnp.maximum(o, 0.0)
    gT = lax.dot_general(
        w2_ref[...], h1,
        dimension_numbers=(((0,), (0,)), ((), ())),
        preferred_element_type=jnp.float32)
    col = jax.lax.broadcasted_iota(jnp.int32, (1, BR), 1) + i * BR
    gsT_ref[...] = jnp.where(col < N, gT * dinvT, 0.0)


def _tc_f(acc_ref, gsT_ref, dinvT_ref, b2_ref, out_ref):
    a = acc_ref[...]
    tot = a[0:8] + a[8:16] + a[16:24] + a[24:32]
    oT = (tot + gsT_ref[...]) * dinvT_ref[...] + b2_ref[...]
    m = jnp.max(oT, axis=0, keepdims=True)
    e = jnp.exp(oT - m)
    lse = jnp.log(jnp.sum(e, axis=0, keepdims=True))
    out_ref[...] = jnp.transpose(oT - m - lse)


def kernel(x, edge_index, W1, b1, W2, b2):
    E = edge_index.shape[1]
    ept = ((E + NW * 2 * CH - 1) // (NW * 2 * CH)) * (2 * CH)
    epad = ept * NW
    pad = epad - E
    n_chunks = ept // CH

    src = jnp.concatenate([edge_index[0], jnp.full((pad,), N, jnp.int32)])
    dst = jnp.concatenate([edge_index[1], jnp.full((pad,), N, jnp.int32)])
    packed = src | (dst << 14)
    dst2 = dst.reshape(NW * n_chunks, CH)

    x_pad = jnp.pad(x, ((0, NPAD - N), (0, 0)))
    b1c = b1.reshape(D, 1)
    b2c = b2.reshape(8, 1)

    deg = _make_deg_kernel(ept)(dst2)
    deg0t = deg[:NPAD].reshape(1, NPAD)
    deg1t = deg[NPAD:].reshape(1, NPAD)

    grid = NPAD // BR
    hsT, hsP, dinvT = pl.pallas_call(
        _tc_b,
        grid=(grid,),
        in_specs=[
            pl.BlockSpec((1, BR), lambda i: (0, i)),
            pl.BlockSpec((1, BR), lambda i: (0, i)),
            pl.BlockSpec((BR, D), lambda i: (i, 0)),
            pl.BlockSpec((D, D), lambda i: (0, 0)),
        ],
        out_specs=[
            pl.BlockSpec((D, BR), lambda i: (0, i)),
            pl.BlockSpec((D // 2, BR), lambda i: (0, i)),
            pl.BlockSpec((1, BR), lambda i: (0, i)),
        ],
        out_shape=[
            jax.ShapeDtypeStruct((D, NPAD), jnp.float32),
            jax.ShapeDtypeStruct((D // 2, NPAD), jnp.int32),
            jax.ShapeDtypeStruct((1, NPAD), jnp.float32),
        ],
    )(deg0t, deg1t, x_pad, W1)

    accT = _make_colagg_kernel(epad)(packed, hsP)
    a0T = accT[:D]
    a1T = accT[D:]

    gsT = pl.pallas_call(
        _tc_d,
        grid=(grid,),
        in_specs=[
            pl.BlockSpec((D, BR), lambda i: (0, i)),
            pl.BlockSpec((D, BR), lambda i: (0, i)),
            pl.BlockSpec((D, BR), lambda i: (0, i)),
            pl.BlockSpec((1, BR), lambda i: (0, i)),
            pl.BlockSpec((D, 1), lambda i: (0, 0)),
            pl.BlockSpec((D, 8), lambda i: (0, 0)),
        ],
        out_specs=pl.BlockSpec((8, BR), lambda i: (0, i)),
        out_shape=jax.ShapeDtypeStruct((8, NPAD), jnp.float32),
    )(a0T, a1T, hsT, dinvT, b1c, W2)

    acc2 = _make_colagg2_kernel(epad)(packed, gsT)

    out = pl.pallas_call(
        _tc_f,
        grid=(grid,),
        in_specs=[
            pl.BlockSpec((4 * 8, BR), lambda i: (0, i)),
            pl.BlockSpec((8, BR), lambda i: (0, i)),
            pl.BlockSpec((1, BR), lambda i: (0, i)),
            pl.BlockSpec((8, 1), lambda i: (0, 0)),
        ],
        out_specs=pl.BlockSpec((BR, 8), lambda i: (i, 0)),
        out_shape=jax.ShapeDtypeStruct((NPAD, 8), jnp.float32),
    )(acc2, gsT, dinvT, b2c)

    return out[:N]


# speedup vs baseline: 1.2185x; 1.0847x over previous
import functools

import jax
import jax.numpy as jnp
from jax import lax
from jax.experimental import pallas as pl
from jax.experimental.pallas import tpu as pltpu
from jax.experimental.pallas import tpu_sc as plsc

N = 10000
NPAD = 10240
D = 128
NC, NS = 2, 16
NW = NC * NS
CH = 128
EK = 1024
CPT = 8
BR = 1024

_MESH = plsc.VectorSubcoreMesh(core_axis_name="c", subcore_axis_name="s")
_SC_PARAMS = pltpu.CompilerParams(
    use_tc_tiling_on_sc=False, needs_layout_passes=False)


def _zero_vmem_2d(ref, rows, cols):
    zc = cols // 16

    def body(i, _):
        r = i // zc
        k = i % zc
        ref[r, pl.ds(k * 16, 16)] = jnp.zeros((16,), jnp.float32)
        return 0

    lax.fori_loop(0, rows * zc, body, 0)


def _make_deg_kernel(ept):
    n_chunks = ept // CH
    grp = 16

    @functools.partial(
        pl.kernel,
        out_type=jax.ShapeDtypeStruct((NC * NPAD,), jnp.float32),
        mesh=_MESH,
        compiler_params=_SC_PARAMS,
        scratch_types=[
            pltpu.VMEM((n_chunks, CH), jnp.int32),
            pltpu.VMEM((CH,), jnp.float32),
            pltpu.VMEM((NPAD // NS,), jnp.float32),
            pltpu.VMEM_SHARED((NPAD,), jnp.float32),
            pltpu.SemaphoreType.DMA,
            pltpu.SemaphoreType.DMA,
        ],
    )
    def deg_kernel(dst_hbm, deg_hbm, idx_v, ones_v, stage_v, acc_sh,
                   sem_i, sem_s):
        c = lax.axis_index("c")
        s = lax.axis_index("s")
        wid = c * NS + s
        seg = NPAD // NS

        idx_dma = pltpu.async_copy(
            dst_hbm.at[pl.ds(wid * n_chunks, n_chunks)], idx_v, sem_i)

        def zbody(i, _):
            stage_v[pl.ds(i * 16, 16)] = jnp.zeros((16,), jnp.float32)
            return 0

        lax.fori_loop(0, seg // 16, zbody, 0)

        def obody(i, _):
            ones_v[pl.ds(i * 16, 16)] = jnp.ones((16,), jnp.float32)
            return 0

        lax.fori_loop(0, CH // 16, obody, 0)

        pltpu.sync_copy(stage_v, acc_sh.at[pl.ds(s * seg, seg)])
        plsc.subcore_barrier()
        idx_dma.wait()

        def group(g, _):
            def fire(j, _):
                pltpu.async_copy(
                    ones_v, acc_sh.at[idx_v.at[g * grp + j]], sem_s, add=True)
                return 0

            lax.fori_loop(0, grp, fire, 0)

            def drain(j, _):
                pltpu.make_async_copy(
                    ones_v, acc_sh.at[idx_v.at[0]], sem_s).wait()
                return 0

            lax.fori_loop(0, grp, drain, 0)
            return 0

        lax.fori_loop(0, n_chunks // grp, group, 0)
        plsc.subcore_barrier()

        pltpu.sync_copy(acc_sh.at[pl.ds(s * seg, seg)], stage_v)
        pltpu.sync_copy(stage_v, deg_hbm.at[pl.ds(c * NPAD + s * seg, seg)])

    return deg_kernel


def _make_colagg_kernel(epad):
    eps = epad // NC
    n_ck = eps // EK
    CP2 = CPT // 2

    @functools.partial(
        pl.kernel,
        out_type=jax.ShapeDtypeStruct((NC * D, NPAD), jnp.float32),
        mesh=_MESH,
        compiler_params=_SC_PARAMS,
        scratch_types=[
            pltpu.VMEM((CP2, NPAD), jnp.int32),
            pltpu.VMEM((CPT, NPAD), jnp.float32),
            pltpu.VMEM((EK,), jnp.int32),
            pltpu.VMEM((EK,), jnp.int32),
            pltpu.SemaphoreType.DMA,
            pltpu.SemaphoreType.DMA,
            pltpu.SemaphoreType.DMA,
        ],
    )
    def colagg_kernel(pk_hbm, hsP_hbm, accT_hbm,
                      tbl, acc, e0, e1, sem_t, sem0, sem1):
        c = lax.axis_index("c")
        s = lax.axis_index("s")
        ebase = c * eps

        def compute(buf):
            @plsc.parallel_loop(0, EK // 16, 1, unroll=4)
            def grp(g):
                ev = buf[pl.ds(g * 16, 16)]
                sv = ev & 0x3FFF
                dv = lax.shift_right_logical(ev, 14)
                for j in range(CP2):
                    jj = jnp.full((16,), j, jnp.int32)
                    vp = plsc.load_gather(tbl, [jj, sv])
                    lo = plsc.bitcast(lax.shift_left(vp, 16), jnp.float32)
                    hi = plsc.bitcast(vp & jnp.int32(-65536), jnp.float32)
                    j1 = jnp.full((16,), CP2 + j, jnp.int32)
                    plsc.addupdate_scatter(acc, [jj, dv], lo)
                    plsc.addupdate_scatter(acc, [j1, dv], hi)

        tdma = pltpu.async_copy(
            hsP_hbm.at[pl.ds(s * CP2, CP2)], tbl, sem_t)
        _zero_vmem_2d(acc, CPT, NPAD)
        tdma.wait()

        pltpu.async_copy(pk_hbm.at[pl.ds(ebase, EK)], e0, sem0)

        def pair(i, _):
            c0 = 2 * i
            pltpu.async_copy(
                pk_hbm.at[pl.ds(ebase + (c0 + 1) * EK, EK)], e1, sem1)
            pltpu.make_async_copy(
                pk_hbm.at[pl.ds(0, EK)], e0, sem0).wait()
            compute(e0)

            @pl.when(i < n_ck // 2 - 1)
            def _():
                pltpu.async_copy(
                    pk_hbm.at[pl.ds(ebase + (c0 + 2) * EK, EK)], e0, sem0)

            pltpu.make_async_copy(
                pk_hbm.at[pl.ds(0, EK)], e1, sem1).wait()
            compute(e1)
            return 0

        lax.fori_loop(0, n_ck // 2, pair, 0)

        pltpu.sync_copy(
            acc.at[pl.ds(0, CP2)], accT_hbm.at[pl.ds(c * D + s * CP2, CP2)])
        pltpu.sync_copy(
            acc.at[pl.ds(CP2, CP2)],
            accT_hbm.at[pl.ds(c * D + D // 2 + s * CP2, CP2)])

    return colagg_kernel


def _make_colagg2_kernel(epad):
    eq = epad // 4
    n_ck = eq // EK

    @functools.partial(
        pl.kernel,
        out_type=jax.ShapeDtypeStruct((4 * 8, NPAD), jnp.float32),
        mesh=_MESH,
        compiler_params=_SC_PARAMS,
        scratch_types=[
            pltpu.VMEM((NPAD,), jnp.float32),
            pltpu.VMEM((NPAD,), jnp.float32),
            pltpu.VMEM((EK,), jnp.int32),
            pltpu.VMEM((EK,), jnp.int32),
            pltpu.SemaphoreType.DMA,
            pltpu.SemaphoreType.DMA,
            pltpu.SemaphoreType.DMA,
        ],
    )
    def colagg2_kernel(pk_hbm, gsT_hbm, out_hbm,
                       tbl, acc, e0, e1, sem_t, sem0, sem1):
        c = lax.axis_index("c")
        s = lax.axis_index("s")
        wid = c * NS + s
        col = wid % 8
        q = wid // 8
        ebase = q * eq

        def compute(buf):
            @plsc.parallel_loop(0, EK // 16, 1, unroll=4)
            def grp(g):
                ev = buf[pl.ds(g * 16, 16)]
                sv = ev & 0x3FFF
                dv = lax.shift_right_logical(ev, 14)
                v = plsc.load_gather(tbl, [sv])
                plsc.addupdate_scatter(acc, [dv], v)

        tdma = pltpu.async_copy(gsT_hbm.at[col], tbl, sem_t)

        def zbody(i, _):
            acc[pl.ds(i * 16, 16)] = jnp.zeros((16,), jnp.float32)
            return 0

        lax.fori_loop(0, NPAD // 16, zbody, 0)
        tdma.wait()

        pltpu.async_copy(pk_hbm.at[pl.ds(ebase, EK)], e0, sem0)

        def pair(i, _):
            c0 = 2 * i
            pltpu.async_copy(
                pk_hbm.at[pl.ds(ebase + (c0 + 1) * EK, EK)], e1, sem1)
            pltpu.make_async_copy(
                pk_hbm.at[pl.ds(0, EK)], e0, sem0).wait()
            compute(e0)

            @pl.when(i < n_ck // 2 - 1)
            def _():
                pltpu.async_copy(
                    pk_hbm.at[pl.ds(ebase + (c0 + 2) * EK, EK)], e0, sem0)

            pltpu.make_async_copy(
                pk_hbm.at[pl.ds(0, EK)], e1, sem1).wait()
            compute(e1)
            return 0

        lax.fori_loop(0, n_ck // 2, pair, 0)

        pltpu.sync_copy(acc, out_hbm.at[q * 8 + col])

    return colagg2_kernel


def _tc_b(deg0t_ref, deg1t_ref, x_ref, w1_ref, hsT_ref, hsP_ref, dinvT_ref):
    degT = deg0t_ref[...] + deg1t_ref[...] + 1.0
    dinvT = lax.rsqrt(degT)
    hT = lax.dot_general(
        w1_ref[...], x_ref[...],
        dimension_numbers=(((0,), (1,)), ((), ())),
        preferred_element_type=jnp.float32)
    hsT = hT * dinvT
    hsT_ref[...] = hsT
    dinvT_ref[...] = dinvT
    lo = lax.bitcast_convert_type(
        hsT[:D // 2].astype(jnp.bfloat16), jnp.uint16).astype(jnp.uint32)
    hi = lax.bitcast_convert_type(
        hsT[D // 2:].astype(jnp.bfloat16), jnp.uint16).astype(jnp.uint32)
    hsP_ref[...] = lax.bitcast_convert_type((hi << 16) | lo, jnp.int32)


def _tc_d(a0_ref, a1_ref, hsT_ref, dinvT_ref, b1_ref, w2_ref, gsT_ref):
    i = pl.program_id(0)
    dinvT = dinvT_ref[...]
    pre = a0_ref[...] + a1_ref[...] + hsT_ref[...]
    o = pre * dinvT + b1_ref[...]
    h1 = jnp.maximum(o, 0.0)
    gT = lax.dot_general(
        w2_ref[...], h1,
        dimension_numbers=(((0,), (0,)), ((), ())),
        preferred_element_type=jnp.float32)
    col = jax.lax.broadcasted_iota(jnp.int32, (1, BR), 1) + i * BR
    gsT_ref[...] = jnp.where(col < N, gT * dinvT, 0.0)


def _tc_f(acc_ref, gsT_ref, dinvT_ref, b2_ref, out_ref):
    a = acc_ref[...]
    tot = a[0:8] + a[8:16] + a[16:24] + a[24:32]
    oT = (tot + gsT_ref[...]) * dinvT_ref[...] + b2_ref[...]
    m = jnp.max(oT, axis=0, keepdims=True)
    e = jnp.exp(oT - m)
    lse = jnp.log(jnp.sum(e, axis=0, keepdims=True))
    out_ref[...] = jnp.transpose(oT - m - lse)


def kernel(x, edge_index, W1, b1, W2, b2):
    E = edge_index.shape[1]
    ept = ((E + NW * 2 * CH - 1) // (NW * 2 * CH)) * (2 * CH)
    epad = ept * NW
    pad = epad - E
    n_chunks = ept // CH

    src = jnp.concatenate([edge_index[0], jnp.full((pad,), N, jnp.int32)])
    dst = jnp.concatenate([edge_index[1], jnp.full((pad,), N, jnp.int32)])
    packed = src | (dst << 14)
    dst2 = dst.reshape(NW * n_chunks, CH)

    x_pad = jnp.pad(x, ((0, NPAD - N), (0, 0)))
    b1c = b1.reshape(D, 1)
    b2c = b2.reshape(8, 1)

    deg = _make_deg_kernel(ept)(dst2)
    deg0t = deg[:NPAD].reshape(1, NPAD)
    deg1t = deg[NPAD:].reshape(1, NPAD)

    grid = NPAD // BR
    hsT, hsP, dinvT = pl.pallas_call(
        _tc_b,
        grid=(grid,),
        in_specs=[
            pl.BlockSpec((1, BR), lambda i: (0, i)),
            pl.BlockSpec((1, BR), lambda i: (0, i)),
            pl.BlockSpec((BR, D), lambda i: (i, 0)),
            pl.BlockSpec((D, D), lambda i: (0, 0)),
        ],
        out_specs=[
            pl.BlockSpec((D, BR), lambda i: (0, i)),
            pl.BlockSpec((D // 2, BR), lambda i: (0, i)),
            pl.BlockSpec((1, BR), lambda i: (0, i)),
        ],
        out_shape=[
            jax.ShapeDtypeStruct((D, NPAD), jnp.float32),
            jax.ShapeDtypeStruct((D // 2, NPAD), jnp.int32),
            jax.ShapeDtypeStruct((1, NPAD), jnp.float32),
        ],
    )(deg0t, deg1t, x_pad, W1)

    accT = _make_colagg_kernel(epad)(packed, hsP)
    a0T = accT[:D]
    a1T = accT[D:]

    gsT = pl.pallas_call(
        _tc_d,
        grid=(grid,),
        in_specs=[
            pl.BlockSpec((D, BR), lambda i: (0, i)),
            pl.BlockSpec((D, BR), lambda i: (0, i)),
            pl.BlockSpec((D, BR), lambda i: (0, i)),
            pl.BlockSpec((1, BR), lambda i: (0, i)),
            pl.BlockSpec((D, 1), lambda i: (0, 0)),
            pl.BlockSpec((D, 8), lambda i: (0, 0)),
        ],
        out_specs=pl.BlockSpec((8, BR), lambda i: (0, i)),
        out_shape=jax.ShapeDtypeStruct((8, NPAD), jnp.float32),
    )(a0T, a1T, hsT, dinvT, b1c, W2)

    acc2 = _make_colagg2_kernel(epad)(packed, gsT)

    out = pl.pallas_call(
        _tc_f,
        grid=(grid,),
        in_specs=[
            pl.BlockSpec((4 * 8, BR), lambda i: (0, i)),
            pl.BlockSpec((8, BR), lambda i: (0, i)),
            pl.BlockSpec((1, BR), lambda i: (0, i)),
            pl.BlockSpec((8, 1), lambda i: (0, 0)),
        ],
        out_specs=pl.BlockSpec((BR, 8), lambda i: (i, 0)),
        out_shape=jax.ShapeDtypeStruct((NPAD, 8), jnp.float32),
    )(acc2, gsT, dinvT, b2c)

    return out[:N]
